# Initial kernel scaffold; baseline (speedup 1.0000x reference)
import jax
import jax.numpy as jnp
from jax.experimental import pallas as pl


def kernel(x, W1, b1, W2, b2, W3, b3, W4, b4):
    raise NotImplementedError("write your pallas kernel here")


# speedup vs baseline: 2.4644x; 2.4644x over previous
import functools

import jax
import jax.numpy as jnp
from jax.experimental import pallas as pl
from jax.experimental.pallas import tpu as pltpu
from jax.experimental.pallas import tpu_sc as plsc

N = 10000
D = 512
K = 3
THRESH = 0.9
NP = 10240
RB = 256
CB = 1024
BIG = 1e30
IBIG = 2**30


def _knn_body(xr_ref, xc_ref, sqc_ref, i0_ref, i1_ref, i2_ref,
              g0_ref, g1_ref, g2_ref,
              vs0, vs1, vs2, is0, is1, is2, gs0, gs1, gs2):
    j = pl.program_id(1)
    xr = xr_ref[...]
    xc = xc_ref[...]
    g = jax.lax.dot_general(xr, xc, (((1,), (1,)), ((), ())),
                            preferred_element_type=jnp.float32)
    sqr = jnp.sum(xr * xr, axis=1, keepdims=True)
    dist = sqr + sqc_ref[...] - 2.0 * g
    colid = jax.lax.broadcasted_iota(jnp.int32, (RB, CB), 1) + j * CB
    dist = jnp.where(colid >= N, BIG, dist)

    lv, li, lg = [], [], []
    for _ in range(K):
        vt = jnp.min(dist, axis=1, keepdims=True)
        it = jnp.min(jnp.where(dist == vt, colid, IBIG), axis=1, keepdims=True)
        gt = jnp.min(jnp.where(colid == it, g, BIG), axis=1, keepdims=True)
        dist = jnp.where(colid == it, BIG, dist)
        lv.append(vt); li.append(it); lg.append(gt)

    @pl.when(j == 0)
    def _():
        for t, (a, b, c) in enumerate(zip([vs0, vs1, vs2], [is0, is1, is2],
                                          [gs0, gs1, gs2])):
            a[...] = lv[t]; b[...] = li[t]; c[...] = lg[t]

    @pl.when(j > 0)
    def _():
        cv = [vs0[...], vs1[...], vs2[...]] + lv
        ci = [is0[...], is1[...], is2[...]] + li
        cg = [gs0[...], gs1[...], gs2[...]] + lg
        outs = []
        for _ in range(K):
            m = functools.reduce(jnp.minimum, cv)
            mi = functools.reduce(jnp.minimum,
                                  [jnp.where(v == m, i, IBIG)
                                   for v, i in zip(cv, ci)])
            mg = functools.reduce(jnp.minimum,
                                  [jnp.where((v == m) & (i == mi), gg, BIG)
                                   for v, i, gg in zip(cv, ci, cg)])
            cv = [jnp.where((v == m) & (i == mi), BIG, v)
                  for v, i in zip(cv, ci)]
            outs.append((m, mi, mg))
        for t, (a, b, c) in enumerate(zip([vs0, vs1, vs2], [is0, is1, is2],
                                          [gs0, gs1, gs2])):
            a[...] = outs[t][0]; b[...] = outs[t][1]; c[...] = outs[t][2]

    i0_ref[...] = is0[...]; i1_ref[...] = is1[...]; i2_ref[...] = is2[...]
    g0_ref[...] = gs0[...]; g1_ref[...] = gs1[...]; g2_ref[...] = gs2[...]


def _knn(xp, sqc):
    grid = (NP // RB, NP // CB)
    out1 = jax.ShapeDtypeStruct((NP, 1), jnp.int32)
    outf = jax.ShapeDtypeStruct((NP, 1), jnp.float32)
    res = pl.pallas_call(
        _knn_body,
        grid=grid,
        in_specs=[
            pl.BlockSpec((RB, D), lambda i, j: (i, 0)),
            pl.BlockSpec((CB, D), lambda i, j: (j, 0)),
            pl.BlockSpec((1, CB), lambda i, j: (0, j)),
        ],
        out_specs=[pl.BlockSpec((RB, 1), lambda i, j: (i, 0))] * 6,
        out_shape=[out1] * 3 + [outf] * 3,
        scratch_shapes=[pltpu.VMEM((RB, 1), jnp.float32)] * 3
                      + [pltpu.VMEM((RB, 1), jnp.int32)] * 3
                      + [pltpu.VMEM((RB, 1), jnp.float32)] * 3,
        compiler_params=pltpu.CompilerParams(
            dimension_semantics=("parallel", "arbitrary")),
    )(xp, xp, sqc)
    idx = jnp.concatenate(res[:3], axis=1)[:N]
    dots = jnp.concatenate(res[3:], axis=1)[:N]
    return idx, dots


MB = 1000


def _mm_body(h_ref, w_ref, o_ref):
    o_ref[...] = jnp.dot(h_ref[...], w_ref[...],
                         preferred_element_type=jnp.float32)


def _mm(h, w):
    n, d = h.shape
    f = w.shape[1]
    return pl.pallas_call(
        _mm_body,
        grid=(n // MB,),
        in_specs=[pl.BlockSpec((MB, d), lambda i: (i, 0)),
                  pl.BlockSpec((d, f), lambda i: (0, 0))],
        out_specs=pl.BlockSpec((MB, f), lambda i: (i, 0)),
        out_shape=jax.ShapeDtypeStruct((n, f), jnp.float32),
    )(h, w)


def _mm_fused_body(p0_ref, p1_ref, b_ref, w_ref, y_ref, o_ref):
    y = jnp.maximum(p0_ref[...] + p1_ref[...] + b_ref[...], 0.0)
    y_ref[...] = y
    o_ref[...] = jnp.dot(y, w_ref[...], preferred_element_type=jnp.float32)


def _mm_fused(p0, p1, b, w):
    n, d = p0.shape
    f = w.shape[1]
    return pl.pallas_call(
        _mm_fused_body,
        grid=(n // MB,),
        in_specs=[pl.BlockSpec((MB, d), lambda i: (i, 0)),
                  pl.BlockSpec((MB, d), lambda i: (i, 0)),
                  pl.BlockSpec((1, d), lambda i: (0, 0)),
                  pl.BlockSpec((d, f), lambda i: (0, 0))],
        out_specs=[pl.BlockSpec((MB, d), lambda i: (i, 0)),
                   pl.BlockSpec((MB, f), lambda i: (i, 0))],
        out_shape=[jax.ShapeDtypeStruct((n, d), jnp.float32),
                   jax.ShapeDtypeStruct((n, f), jnp.float32)],
    )(p0, p1, b, w)


def _mm_final_body(p0_ref, p1_ref, b3_ref, x1_ref, x2_ref, a1_ref, a2_ref,
                   a3_ref, b4_ref, o_ref):
    x3 = jnp.maximum(p0_ref[...] + p1_ref[...] + b3_ref[...], 0.0)
    o = jnp.dot(x1_ref[...], a1_ref[...], preferred_element_type=jnp.float32)
    o += jnp.dot(x2_ref[...], a2_ref[...], preferred_element_type=jnp.float32)
    o += jnp.dot(x3, a3_ref[...], preferred_element_type=jnp.float32)
    o_ref[...] = o + b4_ref[...]


def _mm_final(p0, p1, b3, x1, x2, a1, a2, a3, b4):
    n = p0.shape[0]
    f = a1.shape[1]
    d1, d2, d3 = a1.shape[0], a2.shape[0], a3.shape[0]
    return pl.pallas_call(
        _mm_final_body,
        grid=(n // MB,),
        in_specs=[pl.BlockSpec((MB, d3), lambda i: (i, 0)),
                  pl.BlockSpec((MB, d3), lambda i: (i, 0)),
                  pl.BlockSpec((1, d3), lambda i: (0, 0)),
                  pl.BlockSpec((MB, d1), lambda i: (i, 0)),
                  pl.BlockSpec((MB, d2), lambda i: (i, 0)),
                  pl.BlockSpec((d1, f), lambda i: (0, 0)),
                  pl.BlockSpec((d2, f), lambda i: (0, 0)),
                  pl.BlockSpec((d3, f), lambda i: (0, 0)),
                  pl.BlockSpec((1, f), lambda i: (0, 0))],
        out_specs=pl.BlockSpec((MB, f), lambda i: (i, 0)),
        out_shape=jax.ShapeDtypeStruct((n, f), jnp.float32),
    )(p0, p1, b3, x1, x2, a1, a2, a3, b4)


EP = 40960
NT = 16
ET = EP // NT
EB = 32
NB = ET // EB
NA = 10112
NROW = NA // NT


def _sc_agg_body(width, edge_split):
    def body(hw_ref, src_ref, dst_ref, ne_ref, zero_ref, out_ref,
             src_v, dst_v, ne_v, rows_v, sem, agg_s):
        c = jax.lax.axis_index("c")
        s = jax.lax.axis_index("s")
        rsl = pl.ds(s * NROW, NROW)
        pltpu.sync_copy(zero_ref.at[rsl], agg_s.at[rsl])
        plsc.subcore_barrier()

        if edge_split:
            nsteps = EP // 2 // NT // EB
            ebase = c * (EP // 2)
        else:
            nsteps = EP // NT // EB
            ebase = 0

        def step(t, carry):
            off = ebase + s * (nsteps * EB) + t * EB
            pltpu.sync_copy(src_ref.at[pl.ds(off, EB)], src_v)
            pltpu.sync_copy(dst_ref.at[pl.ds(off, EB)], dst_v)
            pltpu.sync_copy(ne_ref.at[pl.ds(off * 16, EB * 16)], ne_v)
            if edge_split:
                gsrc = hw_ref.at[src_v]
            else:
                gsrc = hw_ref.at[c].at[src_v]
            pltpu.async_copy(gsrc, rows_v, sem).wait()
            for b in range(EB):
                nb = ne_v[pl.ds(b * 16, 16)]
                for f in range(width // 16):
                    sl = pl.ds(f * 16, 16)
                    rows_v[b, sl] = rows_v[b, sl] * nb
            pltpu.sync_copy(rows_v, agg_s.at[dst_v], add=True)
            return carry

        jax.lax.fori_loop(0, nsteps, step, 0)
        plsc.subcore_barrier()
        pltpu.sync_copy(agg_s.at[rsl], out_ref.at[c].at[rsl])
    return body


def _sc_agg_call(hw_arr, src, dst, ne16, zeros, width, edge_split):
    mesh = plsc.VectorSubcoreMesh(core_axis_name="c", subcore_axis_name="s")
    fn = pl.kernel(
        _sc_agg_body(width, edge_split),
        out_type=jax.ShapeDtypeStruct((2, NA, width), jnp.float32),
        mesh=mesh,
        scratch_types=[
            pltpu.VMEM((EB,), jnp.int32),
            pltpu.VMEM((EB,), jnp.int32),
            pltpu.VMEM((EB * 16,), jnp.float32),
            pltpu.VMEM((EB, width), jnp.float32),
            pltpu.SemaphoreType.DMA,
            pltpu.VMEM_SHARED((NA, width), jnp.float32),
        ],
    )
    return fn(hw_arr, src, dst, ne16, zeros[:, :width])


def _sc_agg_fsplit(hw, src, dst, ne16, zeros):
    f2 = hw.shape[1] // 2
    hw2 = hw.reshape(N, 2, f2).transpose(1, 0, 2)
    out = _sc_agg_call(hw2, src, dst, ne16, zeros, f2, False)
    return out[:, :N].transpose(1, 0, 2).reshape(N, 2 * f2)


def _sc_agg_esplit(hw, src, dst, ne16, zeros):
    out = _sc_agg_call(hw, src, dst, ne16, zeros, hw.shape[1], True)
    return out[0, :N], out[1, :N]


def kernel(x, W1, b1, W2, b2, W3, b3, W4, b4):
    xp = jnp.pad(x, ((0, NP - N), (0, 0)))
    sq = jnp.sum(xp * xp, axis=1)
    idx, dots = _knn(xp, sq[None, :])

    nrm = jnp.maximum(jnp.sqrt(sq[:N]), 1e-12)
    sim = dots / (nrm[:, None] * nrm[idx])
    i_row = jnp.arange(N, dtype=jnp.int32)
    w_e = ((sim > THRESH) & (idx != i_row[:, None])).astype(jnp.float32)

    deg = jnp.ones((N,), jnp.float32).at[idx.reshape(-1)].add(w_e.reshape(-1))
    dinv = jax.lax.rsqrt(deg)
    ne = dinv[:, None] * dinv[idx] * w_e
    selfw = dinv * dinv

    pad = EP - N * (K + 1)
    src = jnp.pad(jnp.concatenate([jnp.repeat(i_row, K), i_row]), (0, pad))
    dst = jnp.pad(jnp.concatenate([idx.reshape(-1), i_row]), (0, pad))
    nea = jnp.pad(jnp.concatenate([ne.reshape(-1), selfw]), (0, pad))
    ne16 = (jnp.broadcast_to(nea[:, None], (EP, 16))
            + jnp.zeros((EP, 16), jnp.float32)).reshape(-1)
    zeros = jnp.zeros((NA, 256), jnp.float32)

    hw1 = _mm(x, W1)
    agg1 = _sc_agg_fsplit(hw1, src, dst, ne16, zeros)
    x1, hw2 = _mm_fused(agg1, zeros[:N], b1[None, :], W2)
    a2p0, a2p1 = _sc_agg_esplit(hw2, src, dst, ne16, zeros)
    x2, hw3 = _mm_fused(a2p0, a2p1, b2[None, :], W3)
    a3p0, a3p1 = _sc_agg_esplit(hw3, src, dst, ne16, zeros)

    W4T = W4.T
    return _mm_final(a3p0, a3p1, b3[None, :], x1, x2,
                     W4T[:256], W4T[256:384], W4T[384:], b4[None, :])
